# baseline (speedup 1.0000x reference)
import functools

import jax
import jax.numpy as jnp
from jax.experimental import pallas as pl


def _proj_body(x_ref, w_ref, p_ref):
    p_ref[0] = jnp.dot(w_ref[...], x_ref[0], preferred_element_type=jnp.float32)


def _pair_body(p1_ref, p2_ref, out_ref, *, tile_i, n):
    t = pl.program_id(1)
    i0 = t * tile_i
    p1_full = p1_ref[0, 0, :]
    p2_full = p2_ref[0, 0, :]
    p1_rows = p1_ref[0, 0, pl.ds(i0, tile_i)]
    p2_rows = p2_ref[0, 0, pl.ds(i0, tile_i)]

    ii = i0 + jax.lax.broadcasted_iota(jnp.int32, (tile_i, n), 0)
    jj = jax.lax.broadcasted_iota(jnp.int32, (tile_i, n), 1)

    upper = ii < jj
    a = jnp.where(upper,
                  p1_rows[:, None] + p2_full[None, :],
                  p1_full[None, :] + p2_rows[:, None])
    s = jax.nn.sigmoid(a)
    out_ref[0] = jnp.where(ii == jj, 0.0, s)


def kernel(node_features, W, b):
    B, F, N = node_features.shape
    Wr = W.reshape(2, F)
    Wb = jnp.stack([jnp.zeros((), W.dtype), b[0]])

    p = pl.pallas_call(
        _proj_body,
        grid=(B,),
        in_specs=[
            pl.BlockSpec((1, F, N), lambda i: (i, 0, 0)),
            pl.BlockSpec((2, F), lambda i: (0, 0)),
        ],
        out_specs=pl.BlockSpec((1, 2, N), lambda i: (i, 0, 0)),
        out_shape=jax.ShapeDtypeStruct((B, 2, N), jnp.float32),
    )(node_features, Wr)
    p = p + Wb[None, :, None]
    p1 = p[:, 0:1, :]
    p2 = p[:, 1:2, :]

    TI = 256
    body = functools.partial(_pair_body, tile_i=TI, n=N)
    out = pl.pallas_call(
        body,
        grid=(B, N // TI),
        in_specs=[
            pl.BlockSpec((1, 1, N), lambda i, t: (i, 0, 0)),
            pl.BlockSpec((1, 1, N), lambda i, t: (i, 0, 0)),
        ],
        out_specs=pl.BlockSpec((1, TI, N), lambda i, t: (i, t, 0)),
        out_shape=jax.ShapeDtypeStruct((B, N, N), jnp.float32),
    )(p1, p2)
    return out


# speedup vs baseline: 1.0210x; 1.0210x over previous
import functools

import jax
import jax.numpy as jnp
from jax.experimental import pallas as pl


def _proj_body(x_ref, w_ref, wb_ref, p_ref):
    p = jnp.dot(w_ref[...], x_ref[0], preferred_element_type=jnp.float32)
    p_ref[0] = jnp.exp(-(p + wb_ref[...]))


def _pair_body(u_ref, v_ref, out_ref, *, tile_i, n):
    t = pl.program_id(1)
    i0 = t * tile_i
    u_full = u_ref[0, 0, :]
    v_full = v_ref[0, 0, :]
    u_rows = u_ref[0, 0, pl.ds(i0, tile_i)]
    v_rows = v_ref[0, 0, pl.ds(i0, tile_i)]

    ii = i0 + jax.lax.broadcasted_iota(jnp.int32, (tile_i, n), 0)
    jj = jax.lax.broadcasted_iota(jnp.int32, (tile_i, n), 1)

    e = jnp.where(ii < jj,
                  u_rows[:, None] * v_full[None, :],
                  u_full[None, :] * v_rows[:, None])
    s = 1.0 / (1.0 + e)
    out_ref[0] = jnp.where(ii == jj, 0.0, s)


def kernel(node_features, W, b):
    B, F, N = node_features.shape
    Wr = W.reshape(2, F)
    Wb = jnp.stack([jnp.zeros((), W.dtype), b[0]]).reshape(2, 1)

    uv = pl.pallas_call(
        _proj_body,
        grid=(B,),
        in_specs=[
            pl.BlockSpec((1, F, N), lambda i: (i, 0, 0)),
            pl.BlockSpec((2, F), lambda i: (0, 0)),
            pl.BlockSpec((2, 1), lambda i: (0, 0)),
        ],
        out_specs=pl.BlockSpec((1, 2, N), lambda i: (i, 0, 0)),
        out_shape=jax.ShapeDtypeStruct((B, 2, N), jnp.float32),
    )(node_features, Wr, Wb)
    u = uv[:, 0:1, :]
    v = uv[:, 1:2, :]

    TI = 256
    body = functools.partial(_pair_body, tile_i=TI, n=N)
    out = pl.pallas_call(
        body,
        grid=(B, N // TI),
        in_specs=[
            pl.BlockSpec((1, 1, N), lambda i, t: (i, 0, 0)),
            pl.BlockSpec((1, 1, N), lambda i, t: (i, 0, 0)),
        ],
        out_specs=pl.BlockSpec((1, TI, N), lambda i, t: (i, t, 0)),
        out_shape=jax.ShapeDtypeStruct((B, N, N), jnp.float32),
    )(u, v)
    return out
